# baseline (speedup 1.0000x reference)
import functools

import jax
import jax.numpy as jnp
from jax import lax
from jax.experimental import pallas as pl
from jax.experimental.pallas import tpu as pltpu
from jax.experimental.pallas import tpu_sc as plsc

_B, _C, _H, _W = 8, 4, 512, 512
_CE, _HE, _WE = 128, 128, 128
_NSEG = 64
_NPIX = _HE * _WE
_KA = 4
_KB = 4
_HEB = _HE // _KA
_HBB = _H // _KB
_NW = 32
_PPW = _NPIX * _B // _NW
_RPW = _PPW // 16


def _sel0(x):
    return x[..., 0]


def _a_body(e_ref, d_ref, s_ref, m_ref, err_ref, seg_ref, mds_ref):
    de = e_ref[0] - d_ref[0]
    err_ref[0] = jnp.sum(de * de, axis=0) * (1.0 / _CE)
    seg_ref[0] = _sel0(s_ref[0, :, 0]).astype(jnp.int32)
    mds_ref[0] = _sel0(m_ref[0, :, 0])


def _a_pass(enc1, dec1, segs5, masks5):
    return pl.pallas_call(
        _a_body,
        grid=(_B, _KA),
        in_specs=[
            pl.BlockSpec((1, _CE, _HEB, _WE), lambda b, k: (b, 0, k, 0)),
            pl.BlockSpec((1, _CE, _HEB, _WE), lambda b, k: (b, 0, k, 0)),
            pl.BlockSpec((1, _HEB, 1, _WE, 4), lambda b, k: (b, k, 0, 0, 0)),
            pl.BlockSpec((1, _HEB, 1, _WE, 4), lambda b, k: (b, k, 0, 0, 0)),
        ],
        out_specs=[
            pl.BlockSpec((1, _HEB, _WE), lambda b, k: (b, k, 0)),
            pl.BlockSpec((1, _HEB, _WE), lambda b, k: (b, k, 0)),
            pl.BlockSpec((1, _HEB, _WE), lambda b, k: (b, k, 0)),
        ],
        out_shape=[
            jax.ShapeDtypeStruct((_B, _HE, _WE), jnp.float32),
            jax.ShapeDtypeStruct((_B, _HE, _WE), jnp.int32),
            jax.ShapeDtypeStruct((_B, _HE, _WE), jnp.float32),
        ],
    )(enc1, dec1, segs5, masks5)


def _b_body(o_ref, i_ref, m_ref, sums_ref, acc_ref):
    b = pl.program_id(0)
    k = pl.program_id(1)

    @pl.when((b == 0) & (k == 0))
    def _init():
        acc_ref[0] = 0.0
        acc_ref[1] = 0.0

    m = m_ref[0, 0]
    o = o_ref[0]
    x = i_ref[0]
    t = jnp.where(m[None] >= 0.5, 0.0, x)
    dd = o - t
    mse = jnp.sum(dd * dd, axis=0)
    w = (m > 0.0).astype(jnp.float32)
    acc_ref[0] += jnp.sum(mse * w)
    acc_ref[1] += jnp.sum(w)

    @pl.when((b == _B - 1) & (k == _KB - 1))
    def _fini():
        sums_ref[0] = acc_ref[0]
        sums_ref[1] = acc_ref[1]


def _b_pass(outputs, inputs, masks):
    return pl.pallas_call(
        _b_body,
        grid=(_B, _KB),
        in_specs=[
            pl.BlockSpec((1, _C, _HBB, _W), lambda b, k: (b, 0, k, 0)),
            pl.BlockSpec((1, _C, _HBB, _W), lambda b, k: (b, 0, k, 0)),
            pl.BlockSpec((1, 1, _HBB, _W), lambda b, k: (b, 0, k, 0)),
        ],
        out_specs=pl.BlockSpec(memory_space=pltpu.SMEM),
        out_shape=jax.ShapeDtypeStruct((2,), jnp.float32),
        scratch_shapes=[pltpu.SMEM((2,), jnp.float32)],
    )(outputs, inputs, masks)


def _sc_body(seg_hbm, err_hbm, mask_hbm, out_hbm, seg_v, err_v, mask_v, table,
             sem):
    c = lax.axis_index("c")
    s = lax.axis_index("s")
    wid = s * 2 + c
    row0 = wid * _RPW

    cp_s = pltpu.async_copy(seg_hbm.at[pl.ds(row0, _RPW)], seg_v, sem)
    cp_e = pltpu.async_copy(err_hbm.at[pl.ds(row0, _RPW)], err_v, sem)
    cp_m = pltpu.async_copy(mask_hbm.at[pl.ds(row0, _RPW)], mask_v, sem)

    zf = jnp.zeros((16,), jnp.float32)
    for r in range(3 * _NSEG):
        table[pl.ds(r * 16, 16)] = zf

    cp_s.wait()
    cp_e.wait()
    cp_m.wait()

    lane = lax.iota(jnp.int32, 16)
    ones_f = jnp.full((16,), 1.0, jnp.float32)

    def step(i):
        sg = seg_v[i]
        e = err_v[i]
        m = mask_v[i]
        pos = jnp.where((m > 0.0) & (m < 0.5), 1.0, 0.0)
        base = sg * 16 + lane
        plsc.addupdate_scatter(table, [base], ones_f)
        plsc.addupdate_scatter(table, [base + (_NSEG * 16)], e)
        plsc.addupdate_scatter(table, [base + (2 * _NSEG * 16)], pos)

    def body(j, carry):
        for u in range(8):
            step(j * 8 + u)
        return carry

    lax.fori_loop(0, _RPW // 8, body, 0)

    pltpu.sync_copy(table, out_hbm.at[wid])


def _sc_segsum(seg2d, err2d, mask2d):
    mesh = plsc.VectorSubcoreMesh(core_axis_name="c", subcore_axis_name="s")
    fn = functools.partial(
        pl.kernel,
        mesh=mesh,
        compiler_params=pltpu.CompilerParams(needs_layout_passes=False),
        out_type=jax.ShapeDtypeStruct((_NW, 3 * _NSEG * 16), jnp.float32),
        scratch_types=[
            pltpu.VMEM((_RPW, 16), jnp.int32),
            pltpu.VMEM((_RPW, 16), jnp.float32),
            pltpu.VMEM((_RPW, 16), jnp.float32),
            pltpu.VMEM((3 * _NSEG * 16,), jnp.float32),
            pltpu.SemaphoreType.DMA,
        ],
    )(_sc_body)
    return fn(seg2d, err2d, mask2d)


def _epi_body(p_ref, s_ref, o_ref):
    t = jnp.sum(p_ref[...], axis=3)
    num = 0.0
    den = 0.0
    for b in range(_B):
        g = t[4 * b] + t[4 * b + 1] + t[4 * b + 2] + t[4 * b + 3]
        counts = g[0]
        errs = g[1]
        pos = g[2]
        cm = jnp.maximum(counts, 1.0)
        mean_err = errs / cm
        valid = (counts / _NPIX) >= 0.01
        is_pos = (pos / cm) > 0.01
        sel = jnp.where(valid & is_pos, 1.0, 0.0)
        num += jnp.sum(mean_err * sel)
        den += jnp.sum(sel)
    o_ref[0] = s_ref[0] / jnp.maximum(s_ref[1], 1.0) + num / jnp.maximum(den, 1.0)


def _epilogue(partials, sums):
    return pl.pallas_call(
        _epi_body,
        in_specs=[
            pl.BlockSpec(memory_space=pltpu.VMEM),
            pl.BlockSpec(memory_space=pltpu.SMEM),
        ],
        out_specs=pl.BlockSpec(memory_space=pltpu.SMEM),
        out_shape=jax.ShapeDtypeStruct((1,), jnp.float32),
    )(partials, sums)


def kernel(outputs, inputs, enc1, dec1, masks, segs, confidence, iteration, epoch):
    segs5 = segs.reshape(_B, _HE, 4, _WE, 4)
    masks5 = masks.reshape(_B, _HE, 4, _WE, 4)

    err, seg_ds, mask_ds = _a_pass(enc1, dec1, segs5, masks5)
    n2 = _B * _NPIX // 16
    partials = _sc_segsum(
        seg_ds.reshape(n2, 16), err.reshape(n2, 16), mask_ds.reshape(n2, 16)
    )
    sums = _b_pass(outputs, inputs, masks)
    loss = _epilogue(partials.reshape(_NW, 3, _NSEG, 16), sums)
    return loss[0]


# speedup vs baseline: 7.8093x; 7.8093x over previous
import functools

import jax
import jax.numpy as jnp
from jax import lax
from jax.experimental import pallas as pl
from jax.experimental.pallas import tpu as pltpu
from jax.experimental.pallas import tpu_sc as plsc

_B, _C, _H, _W = 8, 4, 512, 512
_CE, _HE, _WE = 128, 128, 128
_NSEG = 64
_NPIX = _HE * _WE
_KA = 4
_KB = 4
_HEB = _HE // _KA
_HBB = _H // _KB
_NW = 32
_PPW = _NPIX * _B // _NW
_RPW = _PPW // 16


def _downsample(x):
    rows = x.reshape(_HEB, 4, _W)[:, 0]
    r = lax.broadcasted_iota(jnp.int32, (_W, _WE), 0)
    c = lax.broadcasted_iota(jnp.int32, (_W, _WE), 1)
    sel = (r == c * 4).astype(jnp.float32)
    return jax.lax.dot(rows, sel, precision=jax.lax.Precision.HIGHEST)


def _a_body(e_ref, d_ref, s_ref, m_ref, err_ref, seg_ref, mds_ref):
    de = e_ref[0] - d_ref[0]
    err_ref[0] = jnp.sum(de * de, axis=0) * (1.0 / _CE)
    seg_ref[0] = _downsample(s_ref[0, 0]).astype(jnp.int32)
    mds_ref[0] = _downsample(m_ref[0, 0])


def _a_pass(enc1, dec1, segs, masks):
    return pl.pallas_call(
        _a_body,
        grid=(_B, _KA),
        in_specs=[
            pl.BlockSpec((1, _CE, _HEB, _WE), lambda b, k: (b, 0, k, 0)),
            pl.BlockSpec((1, _CE, _HEB, _WE), lambda b, k: (b, 0, k, 0)),
            pl.BlockSpec((1, 1, 4 * _HEB, _W), lambda b, k: (b, 0, k, 0)),
            pl.BlockSpec((1, 1, 4 * _HEB, _W), lambda b, k: (b, 0, k, 0)),
        ],
        out_specs=[
            pl.BlockSpec((1, _HEB, _WE), lambda b, k: (b, k, 0)),
            pl.BlockSpec((1, _HEB, _WE), lambda b, k: (b, k, 0)),
            pl.BlockSpec((1, _HEB, _WE), lambda b, k: (b, k, 0)),
        ],
        out_shape=[
            jax.ShapeDtypeStruct((_B, _HE, _WE), jnp.float32),
            jax.ShapeDtypeStruct((_B, _HE, _WE), jnp.int32),
            jax.ShapeDtypeStruct((_B, _HE, _WE), jnp.float32),
        ],
    )(enc1, dec1, segs, masks)


def _b_body(o_ref, i_ref, m_ref, sums_ref, acc_ref):
    b = pl.program_id(0)
    k = pl.program_id(1)

    @pl.when((b == 0) & (k == 0))
    def _init():
        acc_ref[0] = 0.0
        acc_ref[1] = 0.0

    m = m_ref[0, 0]
    o = o_ref[0]
    x = i_ref[0]
    t = jnp.where(m[None] >= 0.5, 0.0, x)
    dd = o - t
    mse = jnp.sum(dd * dd, axis=0)
    w = (m > 0.0).astype(jnp.float32)
    acc_ref[0] += jnp.sum(mse * w)
    acc_ref[1] += jnp.sum(w)

    @pl.when((b == _B - 1) & (k == _KB - 1))
    def _fini():
        sums_ref[0] = acc_ref[0]
        sums_ref[1] = acc_ref[1]


def _b_pass(outputs, inputs, masks):
    return pl.pallas_call(
        _b_body,
        grid=(_B, _KB),
        in_specs=[
            pl.BlockSpec((1, _C, _HBB, _W), lambda b, k: (b, 0, k, 0)),
            pl.BlockSpec((1, _C, _HBB, _W), lambda b, k: (b, 0, k, 0)),
            pl.BlockSpec((1, 1, _HBB, _W), lambda b, k: (b, 0, k, 0)),
        ],
        out_specs=pl.BlockSpec(memory_space=pltpu.SMEM),
        out_shape=jax.ShapeDtypeStruct((2,), jnp.float32),
        scratch_shapes=[pltpu.SMEM((2,), jnp.float32)],
    )(outputs, inputs, masks)


def _sc_body(seg_hbm, err_hbm, mask_hbm, out_hbm, seg_v, err_v, mask_v, table,
             sem):
    c = lax.axis_index("c")
    s = lax.axis_index("s")
    wid = s * 2 + c
    row0 = wid * _RPW

    cp_s = pltpu.async_copy(seg_hbm.at[pl.ds(row0, _RPW)], seg_v, sem)
    cp_e = pltpu.async_copy(err_hbm.at[pl.ds(row0, _RPW)], err_v, sem)
    cp_m = pltpu.async_copy(mask_hbm.at[pl.ds(row0, _RPW)], mask_v, sem)

    zf = jnp.zeros((16,), jnp.float32)
    for r in range(3 * _NSEG):
        table[pl.ds(r * 16, 16)] = zf

    cp_s.wait()
    cp_e.wait()
    cp_m.wait()

    lane = lax.iota(jnp.int32, 16)
    ones_f = jnp.full((16,), 1.0, jnp.float32)

    def step(i):
        sg = seg_v[i]
        e = err_v[i]
        m = mask_v[i]
        pos = jnp.where((m > 0.0) & (m < 0.5), 1.0, 0.0)
        base = sg * 16 + lane
        plsc.addupdate_scatter(table, [base], ones_f)
        plsc.addupdate_scatter(table, [base + (_NSEG * 16)], e)
        plsc.addupdate_scatter(table, [base + (2 * _NSEG * 16)], pos)

    def body(j, carry):
        for u in range(8):
            step(j * 8 + u)
        return carry

    lax.fori_loop(0, _RPW // 8, body, 0)

    pltpu.sync_copy(table, out_hbm.at[wid])


def _sc_segsum(seg2d, err2d, mask2d):
    mesh = plsc.VectorSubcoreMesh(core_axis_name="c", subcore_axis_name="s")
    fn = functools.partial(
        pl.kernel,
        mesh=mesh,
        compiler_params=pltpu.CompilerParams(needs_layout_passes=False),
        out_type=jax.ShapeDtypeStruct((_NW, 3 * _NSEG * 16), jnp.float32),
        scratch_types=[
            pltpu.VMEM((_RPW, 16), jnp.int32),
            pltpu.VMEM((_RPW, 16), jnp.float32),
            pltpu.VMEM((_RPW, 16), jnp.float32),
            pltpu.VMEM((3 * _NSEG * 16,), jnp.float32),
            pltpu.SemaphoreType.DMA,
        ],
    )(_sc_body)
    return fn(seg2d, err2d, mask2d)


def _epi_body(p_ref, s_ref, o_ref):
    t = jnp.sum(p_ref[...], axis=3)
    num = 0.0
    den = 0.0
    for b in range(_B):
        g = t[4 * b] + t[4 * b + 1] + t[4 * b + 2] + t[4 * b + 3]
        counts = g[0]
        errs = g[1]
        pos = g[2]
        cm = jnp.maximum(counts, 1.0)
        mean_err = errs / cm
        valid = (counts / _NPIX) >= 0.01
        is_pos = (pos / cm) > 0.01
        sel = jnp.where(valid & is_pos, 1.0, 0.0)
        num += jnp.sum(mean_err * sel)
        den += jnp.sum(sel)
    o_ref[0] = s_ref[0] / jnp.maximum(s_ref[1], 1.0) + num / jnp.maximum(den, 1.0)


def _epilogue(partials, sums):
    return pl.pallas_call(
        _epi_body,
        in_specs=[
            pl.BlockSpec(memory_space=pltpu.VMEM),
            pl.BlockSpec(memory_space=pltpu.SMEM),
        ],
        out_specs=pl.BlockSpec(memory_space=pltpu.SMEM),
        out_shape=jax.ShapeDtypeStruct((1,), jnp.float32),
    )(partials, sums)


def kernel(outputs, inputs, enc1, dec1, masks, segs, confidence, iteration, epoch):
    err, seg_ds, mask_ds = _a_pass(enc1, dec1, segs, masks)
    n2 = _B * _NPIX // 16
    partials = _sc_segsum(
        seg_ds.reshape(n2, 16), err.reshape(n2, 16), mask_ds.reshape(n2, 16)
    )
    sums = _b_pass(outputs, inputs, masks)
    loss = _epilogue(partials.reshape(_NW, 3, _NSEG, 16), sums)
    return loss[0]


# speedup vs baseline: 8.8378x; 1.1317x over previous
import functools

import jax
import jax.numpy as jnp
from jax import lax
from jax.experimental import pallas as pl
from jax.experimental.pallas import tpu as pltpu
from jax.experimental.pallas import tpu_sc as plsc

_B, _C, _H, _W = 8, 4, 512, 512
_CE, _HE, _WE = 128, 128, 128
_NSEG = 64
_NPIX = _HE * _WE
_KD = 4
_HEB = _HE // _KD
_HBB = _H // _KD
_NW = 32
_PPW = _NPIX * _B // _NW
_RPW = _PPW // 16


def _downsample(x):
    rows = x.reshape(_HEB, 4, _W)[:, 0]
    r = lax.broadcasted_iota(jnp.int32, (_W, _WE), 0)
    c = lax.broadcasted_iota(jnp.int32, (_W, _WE), 1)
    sel = (r == c * 4).astype(jnp.float32)
    return jax.lax.dot(rows, sel, precision=jax.lax.Precision.HIGHEST)


def _d_body(o_ref, i_ref, m_ref, e_ref, d_ref, s_ref,
            err_ref, seg_ref, mds_ref, sums_ref, acc_ref):
    b = pl.program_id(0)
    k = pl.program_id(1)

    @pl.when((b == 0) & (k == 0))
    def _init():
        acc_ref[0] = 0.0
        acc_ref[1] = 0.0

    m = m_ref[0, 0]
    o = o_ref[0]
    x = i_ref[0]
    t = jnp.where(m[None] >= 0.5, 0.0, x)
    dd = o - t
    mse = jnp.sum(dd * dd, axis=0)
    w = (m > 0.0).astype(jnp.float32)
    acc_ref[0] += jnp.sum(mse * w)
    acc_ref[1] += jnp.sum(w)

    de = e_ref[0] - d_ref[0]
    err_ref[0] = jnp.sum(de * de, axis=0) * (1.0 / _CE)
    seg_ref[0] = _downsample(s_ref[0, 0]).astype(jnp.int32)
    mds_ref[0] = _downsample(m)

    @pl.when((b == _B - 1) & (k == _KD - 1))
    def _fini():
        sums_ref[0] = acc_ref[0]
        sums_ref[1] = acc_ref[1]


def _dense_pass(outputs, inputs, masks, enc1, dec1, segs):
    return pl.pallas_call(
        _d_body,
        grid=(_B, _KD),
        in_specs=[
            pl.BlockSpec((1, _C, _HBB, _W), lambda b, k: (b, 0, k, 0)),
            pl.BlockSpec((1, _C, _HBB, _W), lambda b, k: (b, 0, k, 0)),
            pl.BlockSpec((1, 1, _HBB, _W), lambda b, k: (b, 0, k, 0)),
            pl.BlockSpec((1, _CE, _HEB, _WE), lambda b, k: (b, 0, k, 0)),
            pl.BlockSpec((1, _CE, _HEB, _WE), lambda b, k: (b, 0, k, 0)),
            pl.BlockSpec((1, 1, _HBB, _W), lambda b, k: (b, 0, k, 0)),
        ],
        out_specs=[
            pl.BlockSpec((1, _HEB, _WE), lambda b, k: (b, k, 0)),
            pl.BlockSpec((1, _HEB, _WE), lambda b, k: (b, k, 0)),
            pl.BlockSpec((1, _HEB, _WE), lambda b, k: (b, k, 0)),
            pl.BlockSpec(memory_space=pltpu.SMEM),
        ],
        out_shape=[
            jax.ShapeDtypeStruct((_B, _HE, _WE), jnp.float32),
            jax.ShapeDtypeStruct((_B, _HE, _WE), jnp.int32),
            jax.ShapeDtypeStruct((_B, _HE, _WE), jnp.float32),
            jax.ShapeDtypeStruct((2,), jnp.float32),
        ],
        scratch_shapes=[pltpu.SMEM((2,), jnp.float32)],
    )(outputs, inputs, masks, enc1, dec1, segs)


def _sc_body(seg_hbm, err_hbm, mask_hbm, out_hbm, seg_v, err_v, mask_v, table,
             sem):
    c = lax.axis_index("c")
    s = lax.axis_index("s")
    wid = s * 2 + c
    row0 = wid * _RPW

    cp_s = pltpu.async_copy(seg_hbm.at[pl.ds(row0, _RPW)], seg_v, sem)
    cp_e = pltpu.async_copy(err_hbm.at[pl.ds(row0, _RPW)], err_v, sem)
    cp_m = pltpu.async_copy(mask_hbm.at[pl.ds(row0, _RPW)], mask_v, sem)

    zf = jnp.zeros((16,), jnp.float32)
    for r in range(3 * _NSEG):
        table[pl.ds(r * 16, 16)] = zf

    cp_s.wait()
    cp_e.wait()
    cp_m.wait()

    lane = lax.iota(jnp.int32, 16)
    ones_f = jnp.full((16,), 1.0, jnp.float32)

    def step(i):
        sg = seg_v[i]
        e = err_v[i]
        m = mask_v[i]
        pos = jnp.where((m > 0.0) & (m < 0.5), 1.0, 0.0)
        base = sg * 16 + lane
        plsc.addupdate_scatter(table, [base], ones_f)
        plsc.addupdate_scatter(table, [base + (_NSEG * 16)], e)
        plsc.addupdate_scatter(table, [base + (2 * _NSEG * 16)], pos)

    def body(j, carry):
        for u in range(8):
            step(j * 8 + u)
        return carry

    lax.fori_loop(0, _RPW // 8, body, 0)

    pltpu.sync_copy(table, out_hbm.at[wid])


def _sc_segsum(seg2d, err2d, mask2d):
    mesh = plsc.VectorSubcoreMesh(core_axis_name="c", subcore_axis_name="s")
    fn = functools.partial(
        pl.kernel,
        mesh=mesh,
        compiler_params=pltpu.CompilerParams(needs_layout_passes=False),
        out_type=jax.ShapeDtypeStruct((_NW, 3 * _NSEG * 16), jnp.float32),
        scratch_types=[
            pltpu.VMEM((_RPW, 16), jnp.int32),
            pltpu.VMEM((_RPW, 16), jnp.float32),
            pltpu.VMEM((_RPW, 16), jnp.float32),
            pltpu.VMEM((3 * _NSEG * 16,), jnp.float32),
            pltpu.SemaphoreType.DMA,
        ],
    )(_sc_body)
    return fn(seg2d, err2d, mask2d)


def _epi_body(p_ref, s_ref, o_ref):
    t = jnp.sum(p_ref[...], axis=3)
    num = 0.0
    den = 0.0
    for b in range(_B):
        g = t[4 * b] + t[4 * b + 1] + t[4 * b + 2] + t[4 * b + 3]
        counts = g[0]
        errs = g[1]
        pos = g[2]
        cm = jnp.maximum(counts, 1.0)
        mean_err = errs / cm
        valid = (counts / _NPIX) >= 0.01
        is_pos = (pos / cm) > 0.01
        sel = jnp.where(valid & is_pos, 1.0, 0.0)
        num += jnp.sum(mean_err * sel)
        den += jnp.sum(sel)
    o_ref[0] = s_ref[0] / jnp.maximum(s_ref[1], 1.0) + num / jnp.maximum(den, 1.0)


def _epilogue(partials, sums):
    return pl.pallas_call(
        _epi_body,
        in_specs=[
            pl.BlockSpec(memory_space=pltpu.VMEM),
            pl.BlockSpec(memory_space=pltpu.SMEM),
        ],
        out_specs=pl.BlockSpec(memory_space=pltpu.SMEM),
        out_shape=jax.ShapeDtypeStruct((1,), jnp.float32),
    )(partials, sums)


def kernel(outputs, inputs, enc1, dec1, masks, segs, confidence, iteration, epoch):
    err, seg_ds, mask_ds, sums = _dense_pass(
        outputs, inputs, masks, enc1, dec1, segs
    )
    n2 = _B * _NPIX // 16
    partials = _sc_segsum(
        seg_ds.reshape(n2, 16), err.reshape(n2, 16), mask_ds.reshape(n2, 16)
    )
    loss = _epilogue(partials.reshape(_NW, 3, _NSEG, 16), sums)
    return loss[0]


# speedup vs baseline: 10.6695x; 1.2073x over previous
import functools

import jax
import jax.numpy as jnp
from jax import lax
from jax.experimental import pallas as pl
from jax.experimental.pallas import tpu as pltpu
from jax.experimental.pallas import tpu_sc as plsc

_B, _C, _H, _W = 8, 4, 512, 512
_CE, _HE, _WE = 128, 128, 128
_NSEG = 64
_NPIX = _HE * _WE
_KD = 4
_HEB = _HE // _KD
_HBB = _H // _KD
_NW = 32
_PPW = _NPIX * _B // _NW
_RPW = _PPW // 16


def _downsample(x):
    rows = x.reshape(_HEB, 4, _W)[:, 0]
    r = lax.broadcasted_iota(jnp.int32, (_W, _WE), 0)
    c = lax.broadcasted_iota(jnp.int32, (_W, _WE), 1)
    sel = (r == c * 4).astype(jnp.float32)
    return jax.lax.dot(rows, sel, precision=jax.lax.Precision.HIGHEST)


def _d_body(o_ref, i_ref, m_ref, e_ref, d_ref, s_ref,
            err_ref, seg_ref, mds_ref, sums_ref, acc_ref):
    b = pl.program_id(0)
    k = pl.program_id(1)

    @pl.when((b == 0) & (k == 0))
    def _init():
        acc_ref[0] = 0.0
        acc_ref[1] = 0.0

    m = m_ref[0, 0]
    o = o_ref[0]
    x = i_ref[0]
    t = jnp.where(m[None] >= 0.5, 0.0, x)
    dd = o - t
    mse = jnp.sum(dd * dd, axis=0)
    w = (m > 0.0).astype(jnp.float32)
    acc_ref[0] += jnp.sum(mse * w)
    acc_ref[1] += jnp.sum(w)

    de = e_ref[0] - d_ref[0]
    err_ref[0] = jnp.sum(de * de, axis=0) * (1.0 / _CE)
    seg_ref[0] = _downsample(s_ref[0, 0]).astype(jnp.int32)
    mds_ref[0] = _downsample(m)

    @pl.when((b == _B - 1) & (k == _KD - 1))
    def _fini():
        sums_ref[0] = acc_ref[0]
        sums_ref[1] = acc_ref[1]


def _dense_pass(outputs, inputs, masks, enc1, dec1, segs):
    return pl.pallas_call(
        _d_body,
        grid=(_B, _KD),
        in_specs=[
            pl.BlockSpec((1, _C, _HBB, _W), lambda b, k: (b, 0, k, 0)),
            pl.BlockSpec((1, _C, _HBB, _W), lambda b, k: (b, 0, k, 0)),
            pl.BlockSpec((1, 1, _HBB, _W), lambda b, k: (b, 0, k, 0)),
            pl.BlockSpec((1, _CE, _HEB, _WE), lambda b, k: (b, 0, k, 0)),
            pl.BlockSpec((1, _CE, _HEB, _WE), lambda b, k: (b, 0, k, 0)),
            pl.BlockSpec((1, 1, _HBB, _W), lambda b, k: (b, 0, k, 0)),
        ],
        out_specs=[
            pl.BlockSpec((1, _HEB, _WE), lambda b, k: (b, k, 0)),
            pl.BlockSpec((1, _HEB, _WE), lambda b, k: (b, k, 0)),
            pl.BlockSpec((1, _HEB, _WE), lambda b, k: (b, k, 0)),
            pl.BlockSpec(memory_space=pltpu.SMEM),
        ],
        out_shape=[
            jax.ShapeDtypeStruct((_B, _HE, _WE), jnp.float32),
            jax.ShapeDtypeStruct((_B, _HE, _WE), jnp.int32),
            jax.ShapeDtypeStruct((_B, _HE, _WE), jnp.float32),
            jax.ShapeDtypeStruct((2,), jnp.float32),
        ],
        scratch_shapes=[pltpu.SMEM((2,), jnp.float32)],
    )(outputs, inputs, masks, enc1, dec1, segs)


_RROWS = _PPW // _WE


def _sc_body(seg_hbm, err_hbm, mask_hbm, out_hbm, seg_v, err_v, mask_v, table,
             sem):
    c = lax.axis_index("c")
    s = lax.axis_index("s")
    wid = s * 2 + c
    row0 = wid * _RROWS

    cp_s = pltpu.async_copy(seg_hbm.at[pl.ds(row0, _RROWS)], seg_v, sem)
    cp_e = pltpu.async_copy(err_hbm.at[pl.ds(row0, _RROWS)], err_v, sem)
    cp_m = pltpu.async_copy(mask_hbm.at[pl.ds(row0, _RROWS)], mask_v, sem)

    zf = jnp.zeros((16,), jnp.float32)
    for r in range(3 * _NSEG):
        table[pl.ds(r * 16, 16)] = zf

    cp_s.wait()
    cp_e.wait()
    cp_m.wait()

    lane = lax.iota(jnp.int32, 16)
    ones_f = jnp.full((16,), 1.0, jnp.float32)

    def step(r, l):
        sg = seg_v[r, pl.ds(l * 16, 16)]
        e = err_v[r, pl.ds(l * 16, 16)]
        m = mask_v[r, pl.ds(l * 16, 16)]
        pos = jnp.where((m > 0.0) & (m < 0.5), 1.0, 0.0)
        base = sg * 16 + lane
        plsc.addupdate_scatter(table, [base], ones_f)
        plsc.addupdate_scatter(table, [base + (_NSEG * 16)], e)
        plsc.addupdate_scatter(table, [base + (2 * _NSEG * 16)], pos)

    def body(r, carry):
        for l in range(_WE // 16):
            step(r, l)
        return carry

    lax.fori_loop(0, _RROWS, body, 0)

    pltpu.sync_copy(table, out_hbm.at[wid])


def _sc_segsum(seg2d, err2d, mask2d):
    mesh = plsc.VectorSubcoreMesh(core_axis_name="c", subcore_axis_name="s")
    fn = functools.partial(
        pl.kernel,
        mesh=mesh,
        compiler_params=pltpu.CompilerParams(needs_layout_passes=False),
        out_type=jax.ShapeDtypeStruct((_NW, 3 * _NSEG * 16), jnp.float32),
        scratch_types=[
            pltpu.VMEM((_RROWS, _WE), jnp.int32),
            pltpu.VMEM((_RROWS, _WE), jnp.float32),
            pltpu.VMEM((_RROWS, _WE), jnp.float32),
            pltpu.VMEM((3 * _NSEG * 16,), jnp.float32),
            pltpu.SemaphoreType.DMA,
        ],
    )(_sc_body)
    return fn(seg2d, err2d, mask2d)


def _epi_body(p_ref, s_ref, o_ref):
    p = p_ref[...]
    r = lax.broadcasted_iota(jnp.int32, (3 * _NSEG * 16, 3 * _NSEG), 0)
    c = lax.broadcasted_iota(jnp.int32, (3 * _NSEG * 16, 3 * _NSEG), 1)
    gm = (r // 16 == c).astype(jnp.float32)
    t = jax.lax.dot(p, gm, precision=jax.lax.Precision.HIGHEST)
    num = 0.0
    den = 0.0
    for b in range(_B):
        g = t[4 * b] + t[4 * b + 1] + t[4 * b + 2] + t[4 * b + 3]
        counts = g[0:_NSEG]
        errs = g[_NSEG:2 * _NSEG]
        pos = g[2 * _NSEG:3 * _NSEG]
        cm = jnp.maximum(counts, 1.0)
        mean_err = errs / cm
        valid = (counts / _NPIX) >= 0.01
        is_pos = (pos / cm) > 0.01
        sel = jnp.where(valid & is_pos, 1.0, 0.0)
        num += jnp.sum(mean_err * sel)
        den += jnp.sum(sel)
    o_ref[0] = s_ref[0] / jnp.maximum(s_ref[1], 1.0) + num / jnp.maximum(den, 1.0)


def _epilogue(partials, sums):
    return pl.pallas_call(
        _epi_body,
        in_specs=[
            pl.BlockSpec(memory_space=pltpu.VMEM),
            pl.BlockSpec(memory_space=pltpu.SMEM),
        ],
        out_specs=pl.BlockSpec(memory_space=pltpu.SMEM),
        out_shape=jax.ShapeDtypeStruct((1,), jnp.float32),
    )(partials, sums)


def kernel(outputs, inputs, enc1, dec1, masks, segs, confidence, iteration, epoch):
    err, seg_ds, mask_ds, sums = _dense_pass(
        outputs, inputs, masks, enc1, dec1, segs
    )
    n2 = _B * _NPIX // _WE
    partials = _sc_segsum(
        seg_ds.reshape(n2, _WE), err.reshape(n2, _WE), mask_ds.reshape(n2, _WE)
    )
    loss = _epilogue(partials, sums)
    return loss[0]


# speedup vs baseline: 10.7744x; 1.0098x over previous
import functools

import jax
import jax.numpy as jnp
from jax import lax
from jax.experimental import pallas as pl
from jax.experimental.pallas import tpu as pltpu
from jax.experimental.pallas import tpu_sc as plsc

_B, _C, _H, _W = 8, 4, 512, 512
_CE, _HE, _WE = 128, 128, 128
_NSEG = 64
_NPIX = _HE * _WE
_KD = 4
_HEB = _HE // _KD
_HBB = _H // _KD
_NW = 32
_PPW = _NPIX * _B // _NW
_RPW = _PPW // 16


def _downsample(x):
    rows = x.reshape(_HEB, 4, _W)[:, 0]
    r = lax.broadcasted_iota(jnp.int32, (_W, _WE), 0)
    c = lax.broadcasted_iota(jnp.int32, (_W, _WE), 1)
    sel = (r == c * 4).astype(jnp.float32)
    return jax.lax.dot(rows, sel, precision=jax.lax.Precision.HIGHEST)


def _d_body(o_ref, i_ref, m_ref, e_ref, d_ref, s_ref,
            err_ref, seg_ref, mds_ref, sums_ref, acc_ref):
    b = pl.program_id(0)
    k = pl.program_id(1)

    @pl.when((b == 0) & (k == 0))
    def _init():
        acc_ref[0] = 0.0
        acc_ref[1] = 0.0

    m = m_ref[0, 0]
    o = o_ref[0]
    x = i_ref[0]
    t = jnp.where(m[None] >= 0.5, 0.0, x)
    dd = o - t
    mse = jnp.sum(dd * dd, axis=0)
    w = (m > 0.0).astype(jnp.float32)
    acc_ref[0] += jnp.sum(mse * w)
    acc_ref[1] += jnp.sum(w)

    de = e_ref[0] - d_ref[0]
    err_ref[0] = jnp.sum(de * de, axis=0) * (1.0 / _CE)
    seg_ref[0] = _downsample(s_ref[0, 0]).astype(jnp.int32)
    mds_ref[0] = _downsample(m)

    @pl.when((b == _B - 1) & (k == _KD - 1))
    def _fini():
        sums_ref[0] = acc_ref[0]
        sums_ref[1] = acc_ref[1]


def _dense_pass(outputs, inputs, masks, enc1, dec1, segs):
    return pl.pallas_call(
        _d_body,
        grid=(_B, _KD),
        in_specs=[
            pl.BlockSpec((1, _C, _HBB, _W), lambda b, k: (b, 0, k, 0)),
            pl.BlockSpec((1, _C, _HBB, _W), lambda b, k: (b, 0, k, 0)),
            pl.BlockSpec((1, 1, _HBB, _W), lambda b, k: (b, 0, k, 0)),
            pl.BlockSpec((1, _CE, _HEB, _WE), lambda b, k: (b, 0, k, 0)),
            pl.BlockSpec((1, _CE, _HEB, _WE), lambda b, k: (b, 0, k, 0)),
            pl.BlockSpec((1, 1, _HBB, _W), lambda b, k: (b, 0, k, 0)),
        ],
        out_specs=[
            pl.BlockSpec((1, _HEB, _WE), lambda b, k: (b, k, 0)),
            pl.BlockSpec((1, _HEB, _WE), lambda b, k: (b, k, 0)),
            pl.BlockSpec((1, _HEB, _WE), lambda b, k: (b, k, 0)),
            pl.BlockSpec(memory_space=pltpu.SMEM),
        ],
        out_shape=[
            jax.ShapeDtypeStruct((_B, _HE, _WE), jnp.float32),
            jax.ShapeDtypeStruct((_B, _HE, _WE), jnp.int32),
            jax.ShapeDtypeStruct((_B, _HE, _WE), jnp.float32),
            jax.ShapeDtypeStruct((2,), jnp.float32),
        ],
        scratch_shapes=[pltpu.SMEM((2,), jnp.float32)],
    )(outputs, inputs, masks, enc1, dec1, segs)


_RROWS = _PPW // _WE


def _sc_body(seg_hbm, err_hbm, mask_hbm, out_hbm, seg_v, err_v, mask_v, table,
             sem):
    c = lax.axis_index("c")
    s = lax.axis_index("s")
    wid = s * 2 + c
    row0 = wid * _RROWS

    cp_s = pltpu.async_copy(seg_hbm.at[pl.ds(row0, _RROWS)], seg_v, sem)
    cp_e = pltpu.async_copy(err_hbm.at[pl.ds(row0, _RROWS)], err_v, sem)
    cp_m = pltpu.async_copy(mask_hbm.at[pl.ds(row0, _RROWS)], mask_v, sem)

    zf = jnp.zeros((16,), jnp.float32)
    for r in range(3 * _NSEG):
        table[pl.ds(r * 16, 16)] = zf

    cp_s.wait()
    cp_e.wait()
    cp_m.wait()

    lane = lax.iota(jnp.int32, 16)
    ones_f = jnp.full((16,), 1.0, jnp.float32)

    def step(r, l):
        sg = seg_v[r, pl.ds(l * 16, 16)]
        e = err_v[r, pl.ds(l * 16, 16)]
        m = mask_v[r, pl.ds(l * 16, 16)]
        pos = jnp.where((m > 0.0) & (m < 0.5), 1.0, 0.0)
        base = sg * 16 + lane
        plsc.addupdate_scatter(table, [base], ones_f)
        plsc.addupdate_scatter(table, [base + (_NSEG * 16)], e)
        plsc.addupdate_scatter(table, [base + (2 * _NSEG * 16)], pos)

    @plsc.parallel_loop(0, _RROWS, unroll=2)
    def _loop(r):
        for l in range(_WE // 16):
            step(r, l)

    pltpu.sync_copy(table, out_hbm.at[wid])


def _sc_segsum(seg2d, err2d, mask2d):
    mesh = plsc.VectorSubcoreMesh(core_axis_name="c", subcore_axis_name="s")
    fn = functools.partial(
        pl.kernel,
        mesh=mesh,
        compiler_params=pltpu.CompilerParams(needs_layout_passes=False),
        out_type=jax.ShapeDtypeStruct((_NW, 3 * _NSEG * 16), jnp.float32),
        scratch_types=[
            pltpu.VMEM((_RROWS, _WE), jnp.int32),
            pltpu.VMEM((_RROWS, _WE), jnp.float32),
            pltpu.VMEM((_RROWS, _WE), jnp.float32),
            pltpu.VMEM((3 * _NSEG * 16,), jnp.float32),
            pltpu.SemaphoreType.DMA,
        ],
    )(_sc_body)
    return fn(seg2d, err2d, mask2d)


def _epi_body(p_ref, s_ref, o_ref):
    p = p_ref[...]
    r = lax.broadcasted_iota(jnp.int32, (3 * _NSEG * 16, 3 * _NSEG), 0)
    c = lax.broadcasted_iota(jnp.int32, (3 * _NSEG * 16, 3 * _NSEG), 1)
    gm = (r // 16 == c).astype(jnp.float32)
    t = jax.lax.dot(p, gm, precision=jax.lax.Precision.HIGHEST)
    num = 0.0
    den = 0.0
    for b in range(_B):
        g = t[4 * b] + t[4 * b + 1] + t[4 * b + 2] + t[4 * b + 3]
        counts = g[0:_NSEG]
        errs = g[_NSEG:2 * _NSEG]
        pos = g[2 * _NSEG:3 * _NSEG]
        cm = jnp.maximum(counts, 1.0)
        mean_err = errs / cm
        valid = (counts / _NPIX) >= 0.01
        is_pos = (pos / cm) > 0.01
        sel = jnp.where(valid & is_pos, 1.0, 0.0)
        num += jnp.sum(mean_err * sel)
        den += jnp.sum(sel)
    o_ref[0] = s_ref[0] / jnp.maximum(s_ref[1], 1.0) + num / jnp.maximum(den, 1.0)


def _epilogue(partials, sums):
    return pl.pallas_call(
        _epi_body,
        in_specs=[
            pl.BlockSpec(memory_space=pltpu.VMEM),
            pl.BlockSpec(memory_space=pltpu.SMEM),
        ],
        out_specs=pl.BlockSpec(memory_space=pltpu.SMEM),
        out_shape=jax.ShapeDtypeStruct((1,), jnp.float32),
    )(partials, sums)


def kernel(outputs, inputs, enc1, dec1, masks, segs, confidence, iteration, epoch):
    err, seg_ds, mask_ds, sums = _dense_pass(
        outputs, inputs, masks, enc1, dec1, segs
    )
    n2 = _B * _NPIX // _WE
    partials = _sc_segsum(
        seg_ds.reshape(n2, _WE), err.reshape(n2, _WE), mask_ds.reshape(n2, _WE)
    )
    loss = _epilogue(partials, sums)
    return loss[0]
